# baseline (speedup 1.0000x reference)
import functools

import jax
import jax.numpy as jnp
from jax import lax
from jax.experimental import pallas as pl
from jax.experimental.pallas import tpu as pltpu
from jax.experimental.pallas import tpu_sc as plsc

EMB = 128
NUM_WORDS = 4096
PAIR_NUM = 131072
HALF = PAIR_NUM // 2
L = 16


@functools.cache
def _get_sc_bits(half_idx):
    mesh = plsc.VectorSubcoreMesh(core_axis_name="c", subcore_axis_name="s")
    nw = mesh.num_cores * mesh.num_subcores
    bpw = HALF // nw

    def _sc_bits_body(a_hbm, b_hbm, sm_hbm, out_hbm, a_v, b_v, sm_v, bits_v,
                      sem_a, sem_b, sem_s):
        wid = lax.axis_index("s") * mesh.num_cores + lax.axis_index("c")
        base = half_idx * HALF + wid * bpw
        ca = pltpu.async_copy(a_hbm.at[pl.ds(base, bpw)], a_v, sem_a)
        cb = pltpu.async_copy(b_hbm.at[pl.ds(base, bpw)], b_v, sem_b)
        cs = pltpu.async_copy(sm_hbm, sm_v, sem_s)
        ca.wait()
        cb.wait()
        cs.wait()

        def body(i, carry):
            off = i * L
            va = a_v[pl.ds(off, L)]
            vb = b_v[pl.ds(off, L)]
            sa = plsc.load_gather(sm_v, [va])
            sb = plsc.load_gather(sm_v, [vb])
            bits_v[pl.ds(off, L)] = jnp.where(sa == sb, 1.0, 0.0)
            return carry

        lax.fori_loop(0, bpw // L, body, 0, unroll=4)
        out_base = wid * bpw
        pltpu.sync_copy(bits_v, out_hbm.at[pl.ds(out_base, bpw)])

    return pl.kernel(
        _sc_bits_body,
        out_type=jax.ShapeDtypeStruct((HALF,), jnp.float32),
        mesh=mesh,
        scratch_types=[
            pltpu.VMEM((bpw,), jnp.int32),
            pltpu.VMEM((bpw,), jnp.int32),
            pltpu.VMEM((NUM_WORDS,), jnp.int32),
            pltpu.VMEM((bpw,), jnp.float32),
            pltpu.SemaphoreType.DMA,
            pltpu.SemaphoreType.DMA,
            pltpu.SemaphoreType.DMA,
        ],
        compiler_params=pltpu.CompilerParams(needs_layout_passes=False),
        name=f"sc_same_speaker_bits_h{half_idx}",
    )


_P = 8192
_Q = _P // 8
_HBLK = HALF // _P


def _tc_compute(lp0_ref, genre_ref, bits_ref, gemb_ref, demb_ref, semb_ref, out_ref):
    d = lp0_ref[1] - lp0_ref[0]
    logd = jnp.int32(0)
    for k in range(1, 12):
        logd = logd + jnp.int32(d >= (1 << k))
    log_d = jnp.minimum(logd, 6)
    d_idx = jnp.where(d < 5, d - 1, log_d + 2)
    g = genre_ref[0]

    demb = demb_ref[:]
    di = lax.broadcasted_iota(jnp.int32, demb.shape, 0)
    drow = jnp.sum(jnp.where(di == d_idx, demb, 0.0), axis=0, keepdims=True)

    gemb = gemb_ref[:]
    gi = lax.broadcasted_iota(jnp.int32, gemb.shape, 0)
    grow = jnp.sum(jnp.where(gi == g, gemb, 0.0), axis=0, keepdims=True)

    se = semb_ref[:]
    base0 = se[0:1, :].reshape(1, 1, EMB)
    diff = (se[1:2, :] - se[0:1, :]).reshape(1, 1, EMB)

    bits = bits_ref[:].reshape(_Q, 8, 1)
    out_ref[:, :, 0:EMB] = base0 + bits * diff
    out_ref[:, :, EMB:2 * EMB] = jnp.broadcast_to(drow.reshape(1, 1, EMB), (_Q, 8, EMB))
    out_ref[:, :, 2 * EMB:3 * EMB] = jnp.broadcast_to(grow.reshape(1, 1, EMB), (_Q, 8, EMB))


def _tc_body_a(lp0_ref, genre_ref, bits_ref, gemb_ref, demb_ref, semb_ref, out_ref):
    _tc_compute(lp0_ref, genre_ref, bits_ref, gemb_ref, demb_ref, semb_ref, out_ref)


def _tc_body_b(lp0_ref, genre_ref, bits_ref, gemb_ref, demb_ref, semb_ref, buf_ref,
               out_ref):
    del buf_ref
    _tc_compute(lp0_ref, genre_ref, bits_ref, gemb_ref, demb_ref, semb_ref, out_ref)


def kernel(link_pairs, speaker_map, genre, genre_emb, distance_emb, speaker_emb):
    a = link_pairs[:, 0].astype(jnp.int32)
    b = link_pairs[:, 1].astype(jnp.int32)
    sm = speaker_map.astype(jnp.int32)
    bits0 = _get_sc_bits(0)(a, b, sm)
    bits1 = _get_sc_bits(1)(a, b, sm)
    bits0_2d = bits0.reshape(HALF // 8, 8)
    bits1_2d = bits1.reshape(HALF // 8, 8)
    lp0 = link_pairs[0, :].astype(jnp.int32)
    genre_arr = jnp.asarray(genre, jnp.int32).reshape(1)

    out_shape = jax.ShapeDtypeStruct((PAIR_NUM // 8, 8, 3 * EMB), jnp.float32)
    cparams = pltpu.CompilerParams(vmem_limit_bytes=110 * 1024 * 1024)
    table_specs = [
        pl.BlockSpec((7, EMB), lambda i: (0, 0)),
        pl.BlockSpec((9, EMB), lambda i: (0, 0)),
        pl.BlockSpec((2, EMB), lambda i: (0, 0)),
    ]

    buf = pl.pallas_call(
        _tc_body_a,
        grid=(_HBLK,),
        in_specs=[
            pl.BlockSpec(memory_space=pltpu.SMEM),
            pl.BlockSpec(memory_space=pltpu.SMEM),
            pl.BlockSpec((_Q, 8), lambda i: (i, 0)),
            *table_specs,
        ],
        out_specs=pl.BlockSpec((_Q, 8, 3 * EMB), lambda i: (i, 0, 0)),
        out_shape=out_shape,
        compiler_params=cparams,
    )(lp0, genre_arr, bits0_2d, genre_emb, distance_emb, speaker_emb)

    out3 = pl.pallas_call(
        _tc_body_b,
        grid=(_HBLK,),
        in_specs=[
            pl.BlockSpec(memory_space=pltpu.SMEM),
            pl.BlockSpec(memory_space=pltpu.SMEM),
            pl.BlockSpec((_Q, 8), lambda i: (i, 0)),
            *table_specs,
            pl.BlockSpec(memory_space=pl.ANY),
        ],
        out_specs=pl.BlockSpec((_Q, 8, 3 * EMB), lambda i: (i + _HBLK, 0, 0)),
        out_shape=out_shape,
        input_output_aliases={6: 0},
        compiler_params=cparams,
    )(lp0, genre_arr, bits1_2d, genre_emb, distance_emb, speaker_emb, buf)
    return out3.reshape(PAIR_NUM, 3 * EMB)


# speedup vs baseline: 1.0535x; 1.0535x over previous
import functools

import jax
import jax.numpy as jnp
from jax import lax
from jax.experimental import pallas as pl
from jax.experimental.pallas import tpu as pltpu
from jax.experimental.pallas import tpu_sc as plsc

EMB = 128
NUM_WORDS = 4096
PAIR_NUM = 131072
L = 16

@functools.cache
def _get_sc_bits():
    mesh = plsc.VectorSubcoreMesh(core_axis_name="c", subcore_axis_name="s")
    nw = mesh.num_cores * mesh.num_subcores
    bpw = PAIR_NUM // nw

    def _sc_bits_body(a_hbm, b_hbm, sm_hbm, out_hbm, a_v, b_v, sm_v, bits_v,
                      sem_a, sem_b, sem_s):
        wid = lax.axis_index("s") * mesh.num_cores + lax.axis_index("c")
        base = wid * bpw
        ca = pltpu.async_copy(a_hbm.at[pl.ds(base, bpw)], a_v, sem_a)
        cb = pltpu.async_copy(b_hbm.at[pl.ds(base, bpw)], b_v, sem_b)
        cs = pltpu.async_copy(sm_hbm, sm_v, sem_s)
        ca.wait()
        cb.wait()
        cs.wait()

        def body(i, carry):
            off = i * L
            va = a_v[pl.ds(off, L)]
            vb = b_v[pl.ds(off, L)]
            sa = plsc.load_gather(sm_v, [va])
            sb = plsc.load_gather(sm_v, [vb])
            bits_v[pl.ds(off, L)] = jnp.where(sa == sb, 1.0, 0.0)
            return carry

        lax.fori_loop(0, bpw // L, body, 0, unroll=4)
        pltpu.sync_copy(bits_v, out_hbm.at[pl.ds(base, bpw)])

    return pl.kernel(
        _sc_bits_body,
        out_type=jax.ShapeDtypeStruct((PAIR_NUM,), jnp.float32),
        mesh=mesh,
        scratch_types=[
            pltpu.VMEM((bpw,), jnp.int32),
            pltpu.VMEM((bpw,), jnp.int32),
            pltpu.VMEM((NUM_WORDS,), jnp.int32),
            pltpu.VMEM((bpw,), jnp.float32),
            pltpu.SemaphoreType.DMA,
            pltpu.SemaphoreType.DMA,
            pltpu.SemaphoreType.DMA,
        ],
        compiler_params=pltpu.CompilerParams(needs_layout_passes=False),
        name="sc_same_speaker_bits",
    )

_P = 8192
_Q = _P // 8


def _tc_body(lp0_ref, genre_ref, bits_ref, gemb_ref, demb_ref, semb_ref, out_ref):
    d = lp0_ref[1] - lp0_ref[0]
    logd = jnp.int32(0)
    for k in range(1, 12):
        logd = logd + jnp.int32(d >= (1 << k))
    log_d = jnp.minimum(logd, 6)
    d_idx = jnp.where(d < 5, d - 1, log_d + 2)
    g = genre_ref[0]

    demb = demb_ref[:]
    di = lax.broadcasted_iota(jnp.int32, demb.shape, 0)
    drow = jnp.sum(jnp.where(di == d_idx, demb, 0.0), axis=0, keepdims=True)

    gemb = gemb_ref[:]
    gi = lax.broadcasted_iota(jnp.int32, gemb.shape, 0)
    grow = jnp.sum(jnp.where(gi == g, gemb, 0.0), axis=0, keepdims=True)

    se = semb_ref[:]
    base0 = se[0:1, :].reshape(1, 1, EMB)
    diff = (se[1:2, :] - se[0:1, :]).reshape(1, 1, EMB)

    bits = bits_ref[:].reshape(_Q, 8, 1)
    out_ref[:, :, 0:EMB] = base0 + bits * diff
    out_ref[:, :, EMB:2 * EMB] = jnp.broadcast_to(drow.reshape(1, 1, EMB), (_Q, 8, EMB))
    out_ref[:, :, 2 * EMB:3 * EMB] = jnp.broadcast_to(grow.reshape(1, 1, EMB), (_Q, 8, EMB))


def kernel(link_pairs, speaker_map, genre, genre_emb, distance_emb, speaker_emb):
    a = link_pairs[:, 0].astype(jnp.int32)
    b = link_pairs[:, 1].astype(jnp.int32)
    sm = speaker_map.astype(jnp.int32)
    bits = _get_sc_bits()(a, b, sm)
    bits2d = bits.reshape(PAIR_NUM // 8, 8)
    lp0 = link_pairs[0, :].astype(jnp.int32)
    genre_arr = jnp.asarray(genre, jnp.int32).reshape(1)

    grid = PAIR_NUM // _P
    out3 = pl.pallas_call(
        _tc_body,
        grid=(grid,),
        in_specs=[
            pl.BlockSpec(memory_space=pltpu.SMEM),
            pl.BlockSpec(memory_space=pltpu.SMEM),
            pl.BlockSpec((_Q, 8), lambda i: (i, 0)),
            pl.BlockSpec((7, EMB), lambda i: (0, 0)),
            pl.BlockSpec((9, EMB), lambda i: (0, 0)),
            pl.BlockSpec((2, EMB), lambda i: (0, 0)),
        ],
        out_specs=pl.BlockSpec((_Q, 8, 3 * EMB), lambda i: (i, 0, 0)),
        out_shape=jax.ShapeDtypeStruct((PAIR_NUM // 8, 8, 3 * EMB), jnp.float32),
        compiler_params=pltpu.CompilerParams(
            vmem_limit_bytes=110 * 1024 * 1024,
        ),
    )(lp0, genre_arr, bits2d, genre_emb, distance_emb, speaker_emb)
    return out3.reshape(PAIR_NUM, 3 * EMB)
